# baseline (speedup 1.0000x reference)
import jax
import jax.numpy as jnp
from jax import lax
from jax.experimental import pallas as pl
from jax.experimental.pallas import tpu as pltpu
from jax.experimental.pallas import tpu_sc as plsc

B, R, C = 32, 512, 512
W = R + C - 1
CH = 32
HB = B // 2
HR = R // 2
NCH = HR // CH
NG = C // 16


def _zero_buf(buf, zvec):
    def row(i2, _):
        for g in range(W // 16):
            buf[i2, pl.ds(g * 16, 16)] = zvec
        buf[i2, pl.ds(W - 16, 16)] = zvec
        return _
    lax.fori_loop(0, CH, row, None)


def _compute(rb, ci, in_b, out_b, zvec, iota):
    def row(i2, _):
        r = rb + ci * CH + i2
        s = (R - 1) - r
        bf = lax.bitwise_and(s, ~15)
        d = s - bf
        rowv = jnp.full((16,), i2, dtype=jnp.int32)
        t0 = iota - d

        for k in range(1, 5):
            out_b[i2, pl.ds(jnp.minimum(bf + (NG + k) * 16, W - 16), 16)] \
                = zvec

        v = plsc.load_gather(in_b, [rowv, lax.bitwise_and(t0, C - 1)])
        out_b[i2, pl.ds(bf, 16)] = jnp.where(t0 >= 0, v, 0.0)
        for g in range(1, NG):
            v = plsc.load_gather(in_b, [rowv, t0 + g * 16])
            out_b[i2, pl.ds(bf + g * 16, 16)] = v
        t = t0 + NG * 16
        v = plsc.load_gather(in_b, [rowv, lax.bitwise_and(t, C - 1)])
        v = jnp.where(t < C, v, 0.0)
        pos = bf + NG * 16 + iota
        plsc.store_scatter(out_b, [rowv, jnp.minimum(pos, W - 1)], v,
                           mask=pos < W)
        return _

    lax.fori_loop(0, CH, row, None)


def _make_body(half):
    def _body(x_hbm, out_hbm, in0, in1, out0, out1, si0, si1, so0, so1):
        cid = lax.axis_index("c")
        sid = lax.axis_index("s")
        wid = sid * 2 + cid
        b = HB * half + (wid >> 1)
        rb = (wid & 1) * HR
        zvec = jnp.zeros((16,), jnp.float32)
        iota = lax.iota(jnp.int32, 16)
        _zero_buf(out0, zvec)
        _zero_buf(out1, zvec)

        def in_copy(ci, buf, sem):
            return pltpu.make_async_copy(
                x_hbm.at[b, pl.ds(rb + ci * CH, CH), :], buf, sem)

        def out_copy(ci, buf, sem):
            return pltpu.make_async_copy(
                buf, out_hbm.at[wid >> 1, pl.ds(rb + ci * CH, CH), :], sem)

        in_copy(0, in0, si0).start()

        def loop(ci2, _):
            ciA = 2 * ci2
            ciB = ciA + 1
            in_copy(ciB, in1, si1).start()
            in_copy(ciA, in0, si0).wait()

            @pl.when(ci2 > 0)
            def _wa():
                out_copy(ciA - 2, out0, so0).wait()

            _compute(rb, ciA, in0, out0, zvec, iota)
            out_copy(ciA, out0, so0).start()

            @pl.when(ci2 < NCH // 2 - 1)
            def _nb():
                in_copy(ciA + 2, in0, si0).start()

            in_copy(ciB, in1, si1).wait()

            @pl.when(ci2 > 0)
            def _wb():
                out_copy(ciB - 2, out1, so1).wait()

            _compute(rb, ciB, in1, out1, zvec, iota)
            out_copy(ciB, out1, so1).start()
            return _

        lax.fori_loop(0, NCH // 2, loop, None)
        out_copy(NCH - 2, out0, so0).wait()
        out_copy(NCH - 1, out1, so1).wait()

    return _body


def _half_call(half):
    return pl.kernel(
        _make_body(half),
        out_type=jax.ShapeDtypeStruct((HB, R, W), jnp.float32),
        mesh=plsc.VectorSubcoreMesh(core_axis_name="c", subcore_axis_name="s"),
        compiler_params=pltpu.CompilerParams(needs_layout_passes=False),
        scratch_types=[
            pltpu.VMEM((CH, C), jnp.float32),
            pltpu.VMEM((CH, C), jnp.float32),
            pltpu.VMEM((CH, W), jnp.float32),
            pltpu.VMEM((CH, W), jnp.float32),
            pltpu.SemaphoreType.DMA,
            pltpu.SemaphoreType.DMA,
            pltpu.SemaphoreType.DMA,
            pltpu.SemaphoreType.DMA,
        ],
    )


def kernel(input):
    o0 = _half_call(0)(input)
    o1 = _half_call(1)(input)
    return jnp.concatenate([o0, o1], axis=0)


# speedup vs baseline: 1.3600x; 1.3600x over previous
import jax
import jax.numpy as jnp
from jax import lax
from jax.experimental import pallas as pl
from jax.experimental.pallas import tpu as pltpu
from jax.experimental.pallas import tpu_sc as plsc

B, R, C = 32, 512, 512
W = R + C - 1
JP = 128
RC = 128
NP = 8
NK = 4


def _body(x_hbm, out_hbm, in_b, ot0, ot1, zb, si, so0, so1, sz0, sz1):
    cid = lax.axis_index("c")
    sid = lax.axis_index("s")
    wid = sid * 2 + cid
    zvec = jnp.zeros((16,), jnp.float32)
    iota = lax.iota(jnp.int32, 16)
    rows = [iota + 16 * h for h in range(RC // 16)]
    ots = (ot0, ot1)
    sos = (so0, so1)
    szs = (sz0, sz1)

    def zrow(jj, _):
        for h in range(RC // 16):
            zb[jj, pl.ds(16 * h, 16)] = zvec
        return _
    lax.fori_loop(0, JP, zrow, None)

    def in_load(k):
        return pltpu.make_async_copy(
            x_hbm.at[wid, pl.ds(k * RC, RC), :], in_b, si)

    def out_copy(src, p, k, jlen, sem):
        return pltpu.make_async_copy(
            src.at[pl.ds(0, jlen), :],
            out_hbm.at[pl.ds(p * JP, jlen), wid, pl.ds(k * RC, RC)], sem)

    def compute(p, k, ot):
        w0 = JP * (p + k) - (R - 1)

        def col2(jj2, _):
            for u in range(2):
                jj = jj2 * 2 + u
                base = w0 + jj
                for h in range(RC // 16):
                    v = rows[h] + base
                    m = plsc.bitcast(v, jnp.uint32) < jnp.uint32(C)
                    q = lax.bitwise_and(v, C - 1)
                    g = plsc.load_gather(in_b, [rows[h], q])
                    ot[jj, pl.ds(16 * h, 16)] = jnp.where(m, g, 0.0)
            return _

        lax.fori_loop(0, JP // 2, col2, None)

    nb = 0
    nz = 0
    pend_b = []
    pend_z = []
    in_load(0).start()
    for k in range(NK):
        in_load(k).wait()
        if k + 1 < NK:
            prefetched = False
        for p in range(NP):
            jlen = W - p * JP if p == NP - 1 else JP
            if 3 <= p + k <= 7:
                slot = nb & 1
                if nb >= 2:
                    ps, pl_ = pend_b.pop(0)
                    out_copy(ots[ps], 0, 0, pl_, sos[ps]).wait()
                compute(p, k, ots[slot])
                if k + 1 < NK and p + k == 7:
                    in_load(k + 1).start()
                    prefetched = True
                out_copy(ots[slot], p, k, jlen, sos[slot]).start()
                pend_b.append((slot, jlen))
                nb += 1
            else:
                zslot = nz & 1
                if nz >= 2:
                    zs, zl = pend_z.pop(0)
                    out_copy(zb, 0, 0, zl, szs[zs]).wait()
                out_copy(zb, p, k, jlen, szs[zslot]).start()
                pend_z.append((zslot, jlen))
                nz += 1
        if k + 1 < NK and not prefetched:
            in_load(k + 1).start()
    for ps, pl_ in pend_b:
        out_copy(ots[ps], 0, 0, pl_, sos[ps]).wait()
    for zs, zl in pend_z:
        out_copy(zb, 0, 0, zl, szs[zs]).wait()


def kernel(input):
    f = pl.kernel(
        _body,
        out_type=jax.ShapeDtypeStruct((W, B, R), jnp.float32),
        mesh=plsc.VectorSubcoreMesh(core_axis_name="c", subcore_axis_name="s"),
        compiler_params=pltpu.CompilerParams(needs_layout_passes=False),
        scratch_types=[
            pltpu.VMEM((RC, C), jnp.float32),
            pltpu.VMEM((JP, RC), jnp.float32),
            pltpu.VMEM((JP, RC), jnp.float32),
            pltpu.VMEM((JP, RC), jnp.float32),
            pltpu.SemaphoreType.DMA,
            pltpu.SemaphoreType.DMA,
            pltpu.SemaphoreType.DMA,
            pltpu.SemaphoreType.DMA,
            pltpu.SemaphoreType.DMA,
        ],
    )
    out_t = f(input)
    return jnp.transpose(out_t, (1, 2, 0))


# speedup vs baseline: 1.4540x; 1.0691x over previous
import jax
import jax.numpy as jnp
from jax import lax
from jax.experimental import pallas as pl
from jax.experimental.pallas import tpu as pltpu
from jax.experimental.pallas import tpu_sc as plsc

B, R, C = 32, 512, 512
W = R + C - 1
JP = 128
RC = 128
NP = 8
NK = 4
IBW = 640


def _body(x_hbm, out_hbm, in_b, ot0, ot1, zb, si, so0, so1, sz0, sz1):
    cid = lax.axis_index("c")
    sid = lax.axis_index("s")
    wid = sid * 2 + cid
    zvec = jnp.zeros((16,), jnp.float32)
    iota = lax.iota(jnp.int32, 16)
    rows = [iota + 16 * h for h in range(RC // 16)]
    ots = (ot0, ot1)
    sos = (so0, so1)
    szs = (sz0, sz1)

    def in_load(k):
        return pltpu.make_async_copy(
            x_hbm.at[wid, pl.ds(k * RC, RC), :],
            in_b.at[:, pl.ds(0, C)], si)

    in_load(0).start()

    def zflank(rr, _):
        for c in range(C, IBW, 16):
            in_b[rr, pl.ds(c, 16)] = zvec
        return _
    lax.fori_loop(0, RC, zflank, None)

    def zrow(jj, _):
        for h in range(RC // 16):
            zb[jj, pl.ds(16 * h, 16)] = zvec
        return _
    lax.fori_loop(0, JP // 2, zrow, None)

    def out_copy(src, p, k, jlen, sem):
        return pltpu.make_async_copy(
            src.at[pl.ds(0, jlen), :],
            out_hbm.at[pl.ds(p * JP, jlen), wid, pl.ds(k * RC, RC)], sem)

    def out_zero(p, k, joff, jl, sem):
        return pltpu.make_async_copy(
            zb.at[pl.ds(0, jl), :],
            out_hbm.at[pl.ds(p * JP + joff, jl), wid, pl.ds(k * RC, RC)],
            sem)

    def compute(p, k, ot):
        w0 = JP * (p + k) - (R - 1)
        edge = w0 < 0

        def col2(jj2, _):
            s = jj2 * 2 + w0
            for u in range(2):
                for h in range(RC // 16):
                    v = rows[h] + (s + u)
                    if edge:
                        m = v >= 0
                        q = lax.bitwise_and(v, C - 1)
                        g = plsc.load_gather(in_b, [rows[h], q])
                        g = jnp.where(m, g, 0.0)
                    else:
                        g = plsc.load_gather(in_b, [rows[h], v])
                    ot[jj2 * 2 + u, pl.ds(16 * h, 16)] = g
            return _

        lax.fori_loop(0, JP // 2, col2, None)

    nb = 0
    nz = 0
    pend_b = []
    pend_z = []
    for k in range(NK):
        in_load(k).wait()
        for p in range(NP):
            jlen = W - p * JP if p == NP - 1 else JP
            if 3 <= p + k <= 7:
                slot = nb & 1
                if nb >= 2:
                    ps, pl_ = pend_b.pop(0)
                    out_copy(ots[ps], 0, 0, pl_, sos[ps]).wait()
                compute(p, k, ots[slot])
                if k + 1 < NK and p + k == 7:
                    in_load(k + 1).start()
                out_copy(ots[slot], p, k, jlen, sos[slot]).start()
                pend_b.append((slot, jlen))
                nb += 1
            else:
                for joff, jl in ((0, JP // 2), (JP // 2, jlen - JP // 2)):
                    zslot = nz & 1
                    if nz >= 2:
                        zs, zl = pend_z.pop(0)
                        out_zero(0, 0, 0, zl, szs[zs]).wait()
                    out_zero(p, k, joff, jl, szs[zslot]).start()
                    pend_z.append((zslot, jl))
                    nz += 1
    for ps, pl_ in pend_b:
        out_copy(ots[ps], 0, 0, pl_, sos[ps]).wait()
    for zs, zl in pend_z:
        out_zero(0, 0, 0, zl, szs[zs]).wait()


def kernel(input):
    f = pl.kernel(
        _body,
        out_type=jax.ShapeDtypeStruct((W, B, R), jnp.float32),
        mesh=plsc.VectorSubcoreMesh(core_axis_name="c", subcore_axis_name="s"),
        compiler_params=pltpu.CompilerParams(needs_layout_passes=False),
        scratch_types=[
            pltpu.VMEM((RC, IBW), jnp.float32),
            pltpu.VMEM((JP, RC), jnp.float32),
            pltpu.VMEM((JP, RC), jnp.float32),
            pltpu.VMEM((JP // 2, RC), jnp.float32),
            pltpu.SemaphoreType.DMA,
            pltpu.SemaphoreType.DMA,
            pltpu.SemaphoreType.DMA,
            pltpu.SemaphoreType.DMA,
            pltpu.SemaphoreType.DMA,
        ],
    )
    out_t = f(input)
    return jnp.transpose(out_t, (1, 2, 0))
